# Initial kernel scaffold; baseline (speedup 1.0000x reference)
import jax
import jax.numpy as jnp
from jax.experimental import pallas as pl


def kernel(x, edge_index, W1, att_src1, att_dst1, b1, W2, att_src2, att_dst2, b2):
    raise NotImplementedError("write your pallas kernel here")


# speedup vs baseline: 1.1586x; 1.1586x over previous
import jax
import jax.numpy as jnp
from jax.experimental import pallas as pl

N = 10000
HEADS = 8
N_UNITS = 32
OUT_CH = 64


def _log_softmax_body(x_ref, o_ref):
    x = x_ref[...]
    m = jnp.max(x, axis=-1, keepdims=True)
    e = jnp.exp(x - m)
    o_ref[...] = (x - m) - jnp.log(jnp.sum(e, axis=-1, keepdims=True))


def _gat_conv(x, edge_index, W, att_src, att_dst, bias, heads, out_ch):
    n = x.shape[0]
    loop = jnp.arange(n, dtype=edge_index.dtype)
    src = jnp.concatenate([edge_index[0], loop])
    dst = jnp.concatenate([edge_index[1], loop])
    h = (x @ W).reshape(n, heads, out_ch)
    a_src = (h * att_src[None, :, :]).sum(-1)
    a_dst = (h * att_dst[None, :, :]).sum(-1)
    alpha = a_src[src] + a_dst[dst]
    alpha = jax.nn.leaky_relu(alpha, 0.2)
    w = jnp.exp(alpha)
    denom = jax.ops.segment_sum(w, dst, num_segments=n)
    msg = h[src] * w[:, :, None]
    out = jax.ops.segment_sum(msg, dst, num_segments=n)
    out = out / (denom + 1e-16)[:, :, None]
    return out.reshape(n, heads * out_ch) + bias


def kernel(x, edge_index, W1, att_src1, att_dst1, b1, W2, att_src2, att_dst2, b2):
    h = _gat_conv(x, edge_index, W1, att_src1, att_dst1, b1, HEADS, N_UNITS)
    h = jax.nn.relu(h)
    out = _gat_conv(h, edge_index, W2, att_src2, att_dst2, b2, 1, OUT_CH)
    return pl.pallas_call(
        _log_softmax_body,
        out_shape=jax.ShapeDtypeStruct((N, OUT_CH), jnp.float32),
    )(out)


# speedup vs baseline: 18.7782x; 16.2075x over previous
import functools

import jax
import jax.numpy as jnp
from jax import lax
from jax.experimental import pallas as pl
from jax.experimental.pallas import tpu as pltpu
from jax.experimental.pallas import tpu_sc as plsc

N = 10000
E = 320000
IN_CH = 128
N_UNITS = 32
HEADS = 8
OUT_CH = 64

K = 128
EP = 331776
PER_TILE1 = EP // 16
NCH1 = PER_TILE1 // K
PER_TILE2 = EP // 32
NCH2 = PER_TILE2 // K

ACC1_R = 40064
DEN_R = 10112
ACC2_R = 10112

_MESH = plsc.VectorSubcoreMesh(core_axis_name="c", subcore_axis_name="s")


def _edge_weights(sa, da, wbuf):
    def body(e, c):
        s = sa[e, :] + da[e, :]
        wbuf[e, :] = jnp.exp(jnp.maximum(s, 0.2 * s))
        return c
    lax.fori_loop(0, K, body, 0, unroll=4)


def _l1_body(src_r, dst_r, as_r, ad_r, h1t_r, zacc_r, zden_r,
             acc_o, den_o,
             acc_sp, den_sp, sidx, didx, gidx, sa, da, wbuf, hrows):
    core = lax.axis_index("c")
    sub = lax.axis_index("s")

    pltpu.sync_copy(zacc_r, acc_sp.at[pl.ds(sub * 2504, 2504)])
    pltpu.sync_copy(zden_r, den_sp.at[pl.ds(sub * 632, 632)])
    plsc.subcore_barrier()

    def chunk(ci, c):
        base = sub * PER_TILE1 + ci * K
        pltpu.sync_copy(src_r.at[pl.ds(base, K)], sidx)
        pltpu.sync_copy(dst_r.at[pl.ds(base, K)], didx)
        toff = core * (N + 1)
        def aib(g, cc):
            gidx[pl.ds(g * 16, 16)] = sidx[pl.ds(g * 16, 16)] + toff
            return cc
        lax.fori_loop(0, K // 16, aib, 0, unroll=True)
        pltpu.sync_copy(as_r.at[gidx], sa)
        def bib(g, cc):
            gidx[pl.ds(g * 16, 16)] = didx[pl.ds(g * 16, 16)] + toff
            return cc
        lax.fori_loop(0, K // 16, bib, 0, unroll=True)
        pltpu.sync_copy(ad_r.at[gidx], da)
        _edge_weights(sa, da, wbuf)
        pltpu.sync_copy(wbuf, den_sp.at[didx], add=True)
        for j in range(4):
            hg = core * 4 + j
            def gib(g, cc):
                gidx[pl.ds(g * 16, 16)] = sidx[pl.ds(g * 16, 16)] * 8 + hg
                return cc
            lax.fori_loop(0, K // 16, gib, 0, unroll=True)
            pltpu.sync_copy(h1t_r.at[gidx], hrows)
            def mul(e, cc):
                wv = jnp.full((16,), wbuf[e, :][j], jnp.float32)
                hrows[e, pl.ds(0, 16)] = hrows[e, pl.ds(0, 16)] * wv
                hrows[e, pl.ds(16, 16)] = hrows[e, pl.ds(16, 16)] * wv
                return cc
            lax.fori_loop(0, K, mul, 0, unroll=4)
            def sib(g, cc):
                gidx[pl.ds(g * 16, 16)] = didx[pl.ds(g * 16, 16)] * 4 + j
                return cc
            lax.fori_loop(0, K // 16, sib, 0, unroll=True)
            pltpu.sync_copy(hrows, acc_sp.at[gidx], add=True)
        return c

    lax.fori_loop(0, NCH1, chunk, 0)
    plsc.subcore_barrier()

    pltpu.sync_copy(acc_sp.at[pl.ds(sub * 2504, 2504)],
                    acc_o.at[core, pl.ds(sub * 2504, 2504)])

    @pl.when(core == 0)
    def _():
        pltpu.sync_copy(den_sp.at[pl.ds(sub * 632, 632)],
                        den_o.at[pl.ds(sub * 632, 632)])


def _l2_body(src_r, dst_r, as_r, ad_r, h2t_r, zacc_r, zden_r,
             acc_o, den_o,
             acc_sp, den_sp, sidx, didx, sa, da, wbuf, hrows):
    core = lax.axis_index("c")
    sub = lax.axis_index("s")

    pltpu.sync_copy(zacc_r, acc_sp.at[pl.ds(sub * 632, 632)])
    pltpu.sync_copy(zden_r, den_sp.at[pl.ds(sub * 632, 632)])
    plsc.subcore_barrier()

    def chunk(ci, c):
        base = core * (EP // 2) + sub * PER_TILE2 + ci * K
        pltpu.sync_copy(src_r.at[pl.ds(base, K)], sidx)
        pltpu.sync_copy(dst_r.at[pl.ds(base, K)], didx)
        pltpu.sync_copy(as_r.at[sidx], sa)
        pltpu.sync_copy(ad_r.at[didx], da)
        _edge_weights(sa, da, wbuf)
        pltpu.sync_copy(wbuf, den_sp.at[didx], add=True)
        pltpu.sync_copy(h2t_r.at[sidx], hrows)
        def mul(e, cc):
            wv = jnp.full((16,), wbuf[e, :][0], jnp.float32)
            for q in range(4):
                hrows[e, pl.ds(16 * q, 16)] = hrows[e, pl.ds(16 * q, 16)] * wv
            return cc
        lax.fori_loop(0, K, mul, 0, unroll=4)
        pltpu.sync_copy(hrows, acc_sp.at[didx], add=True)
        return c

    lax.fori_loop(0, NCH2, chunk, 0)
    plsc.subcore_barrier()

    pltpu.sync_copy(acc_sp.at[pl.ds(sub * 632, 632)],
                    acc_o.at[core, pl.ds(sub * 632, 632)])
    pltpu.sync_copy(den_sp.at[pl.ds(sub * 632, 632)],
                    den_o.at[core, pl.ds(sub * 632, 632)])


_sc_l1 = pl.kernel(
    _l1_body,
    out_type=[
        jax.ShapeDtypeStruct((2, 40064, 32), jnp.float32),
        jax.ShapeDtypeStruct((10112, 16), jnp.float32),
    ],
    mesh=_MESH,
    compiler_params=pltpu.CompilerParams(use_tc_tiling_on_sc=False),
    scratch_types=[
        pltpu.VMEM_SHARED((ACC1_R, 32), jnp.float32),
        pltpu.VMEM_SHARED((DEN_R, 16), jnp.float32),
        pltpu.VMEM((K,), jnp.int32),
        pltpu.VMEM((K,), jnp.int32),
        pltpu.VMEM((K,), jnp.int32),
        pltpu.VMEM((K, 16), jnp.float32),
        pltpu.VMEM((K, 16), jnp.float32),
        pltpu.VMEM((K, 16), jnp.float32),
        pltpu.VMEM((K, 32), jnp.float32),
    ],
)

_sc_l2 = pl.kernel(
    _l2_body,
    out_type=[
        jax.ShapeDtypeStruct((2, 10112, 64), jnp.float32),
        jax.ShapeDtypeStruct((2, 10112, 16), jnp.float32),
    ],
    mesh=_MESH,
    compiler_params=pltpu.CompilerParams(use_tc_tiling_on_sc=False),
    scratch_types=[
        pltpu.VMEM_SHARED((ACC2_R, 64), jnp.float32),
        pltpu.VMEM_SHARED((DEN_R, 16), jnp.float32),
        pltpu.VMEM((K,), jnp.int32),
        pltpu.VMEM((K,), jnp.int32),
        pltpu.VMEM((K, 16), jnp.float32),
        pltpu.VMEM((K, 16), jnp.float32),
        pltpu.VMEM((K, 16), jnp.float32),
        pltpu.VMEM((K, 64), jnp.float32),
    ],
)


_BLK = 1000


def _tc1_body(x_ref, w1_ref, asp_ref, adp_ref, h_ref, a_ref, b_ref):
    h = x_ref[...] @ w1_ref[...]
    h_ref[...] = h
    a_ref[...] = h @ asp_ref[...]
    b_ref[...] = h @ adp_ref[...]


_tc1 = pl.pallas_call(
    _tc1_body,
    grid=(N // _BLK,),
    in_specs=[
        pl.BlockSpec((_BLK, IN_CH), lambda i: (i, 0)),
        pl.BlockSpec((IN_CH, HEADS * N_UNITS), lambda i: (0, 0)),
        pl.BlockSpec((HEADS * N_UNITS, 16), lambda i: (0, 0)),
        pl.BlockSpec((HEADS * N_UNITS, 16), lambda i: (0, 0)),
    ],
    out_specs=[
        pl.BlockSpec((_BLK, HEADS * N_UNITS), lambda i: (i, 0)),
        pl.BlockSpec((_BLK, 16), lambda i: (i, 0)),
        pl.BlockSpec((_BLK, 16), lambda i: (i, 0)),
    ],
    out_shape=[
        jax.ShapeDtypeStruct((N, HEADS * N_UNITS), jnp.float32),
        jax.ShapeDtypeStruct((N, 16), jnp.float32),
        jax.ShapeDtypeStruct((N, 16), jnp.float32),
    ],
)


def _tc2_body(acc_ref, den_ref, e_ref, b1_ref, w2_ref, asp_ref, adp_ref,
              h2_ref, a_ref, b_ref):
    r = 1.0 / (den_ref[...] + 1e-16)
    rexp = r @ e_ref[...]
    h2p = jnp.maximum(acc_ref[...] * rexp + b1_ref[...], 0.0)
    h2 = h2p @ w2_ref[...]
    h2_ref[...] = h2
    a_ref[...] = h2 @ asp_ref[...]
    b_ref[...] = h2 @ adp_ref[...]


_tc2 = pl.pallas_call(
    _tc2_body,
    grid=(N // _BLK,),
    in_specs=[
        pl.BlockSpec((_BLK, HEADS * N_UNITS), lambda i: (i, 0)),
        pl.BlockSpec((_BLK, 16), lambda i: (i, 0)),
        pl.BlockSpec((16, HEADS * N_UNITS), lambda i: (0, 0)),
        pl.BlockSpec((1, HEADS * N_UNITS), lambda i: (0, 0)),
        pl.BlockSpec((HEADS * N_UNITS, OUT_CH), lambda i: (0, 0)),
        pl.BlockSpec((OUT_CH, 16), lambda i: (0, 0)),
        pl.BlockSpec((OUT_CH, 16), lambda i: (0, 0)),
    ],
    out_specs=[
        pl.BlockSpec((_BLK, OUT_CH), lambda i: (i, 0)),
        pl.BlockSpec((_BLK, 16), lambda i: (i, 0)),
        pl.BlockSpec((_BLK, 16), lambda i: (i, 0)),
    ],
    out_shape=[
        jax.ShapeDtypeStruct((N, OUT_CH), jnp.float32),
        jax.ShapeDtypeStruct((N, 16), jnp.float32),
        jax.ShapeDtypeStruct((N, 16), jnp.float32),
    ],
)


def _tc3_body(a0_ref, a1_ref, d0_ref, d1_ref, b2_ref, o_ref):
    den = d0_ref[...][:, 0:1] + d1_ref[...][:, 0:1] + 1e-16
    s = (a0_ref[...] + a1_ref[...]) / den + b2_ref[...]
    m = jnp.max(s, axis=-1, keepdims=True)
    ex = jnp.exp(s - m)
    o_ref[...] = (s - m) - jnp.log(jnp.sum(ex, axis=-1, keepdims=True))


_tc3 = pl.pallas_call(
    _tc3_body,
    grid=(N // _BLK,),
    in_specs=[
        pl.BlockSpec((_BLK, OUT_CH), lambda i: (i, 0)),
        pl.BlockSpec((_BLK, OUT_CH), lambda i: (i, 0)),
        pl.BlockSpec((_BLK, 16), lambda i: (i, 0)),
        pl.BlockSpec((_BLK, 16), lambda i: (i, 0)),
        pl.BlockSpec((1, OUT_CH), lambda i: (0, 0)),
    ],
    out_specs=pl.BlockSpec((_BLK, OUT_CH), lambda i: (i, 0)),
    out_shape=jax.ShapeDtypeStruct((N, OUT_CH), jnp.float32),
)


def kernel(x, edge_index, W1, att_src1, att_dst1, b1, W2, att_src2, att_dst2, b2):
    f32 = jnp.float32
    loop = jnp.arange(N, dtype=jnp.int32)
    padi = jnp.full((EP - E - N,), N, jnp.int32)
    src = jnp.concatenate([edge_index[0], loop, padi])
    dst = jnp.concatenate([edge_index[1], loop, padi])

    hd = jnp.arange(HEADS)
    AsP1 = jnp.zeros((HEADS, N_UNITS, 16), f32).at[hd, :, hd].set(att_src1)
    AsP1 = AsP1.reshape(HEADS * N_UNITS, 16)
    AdP1 = jnp.zeros((HEADS, N_UNITS, 16), f32).at[hd, :, hd].set(att_dst1)
    AdP1 = AdP1.reshape(HEADS * N_UNITS, 16)
    As2P = jnp.zeros((OUT_CH, 16), f32).at[:, 0].set(att_src2[0])
    Ad2P = jnp.zeros((OUT_CH, 16), f32).at[:, 0].set(att_dst2[0])
    E16 = jnp.concatenate(
        [jnp.kron(jnp.eye(HEADS, dtype=f32), jnp.ones((1, N_UNITS), f32)),
         jnp.zeros((8, HEADS * N_UNITS), f32)], axis=0)

    h1, aS1, aD1 = _tc1(x, W1, AsP1, AdP1)

    neg = jnp.full((1, 16), -1e30, f32)

    def _dbl(a, padrow):
        rot = jnp.concatenate([a[:, 4:8], a[:, 0:4], a[:, 8:16]], axis=1)
        return jnp.concatenate([a, padrow, rot, padrow], axis=0)

    aS1t = _dbl(aS1, neg)
    aD1t = _dbl(aD1, jnp.zeros((1, 16), f32))
    h1t = jnp.concatenate(
        [h1.reshape(N * HEADS, N_UNITS),
         jnp.zeros((HEADS, N_UNITS), f32)], axis=0)

    zacc1 = jnp.zeros((2504, 32), f32)
    zden = jnp.zeros((632, 16), f32)
    acc1, den1 = _sc_l1(src, dst, aS1t, aD1t, h1t, zacc1, zden)
    acc1 = acc1[:, :N * 4]
    den1 = den1[:N]

    acc1c = jnp.concatenate([acc1[0].reshape(N, 4 * N_UNITS),
                             acc1[1].reshape(N, 4 * N_UNITS)], axis=1)

    h2, aS2, aD2 = _tc2(acc1c, den1, E16, b1.reshape(1, -1), W2, As2P, Ad2P)

    aS2t = jnp.concatenate([aS2, neg], axis=0)
    aD2t = jnp.concatenate([aD2, jnp.zeros((1, 16), f32)], axis=0)
    h2t = jnp.concatenate([h2, jnp.zeros((1, OUT_CH), f32)], axis=0)

    zacc2 = jnp.zeros((632, 64), f32)
    acc2, den2 = _sc_l2(src, dst, aS2t, aD2t, h2t, zacc2, zden)
    acc2 = acc2[:, :N]
    den2 = den2[:, :N]

    return _tc3(acc2[0], acc2[1], den2[0], den2[1], b2.reshape(1, -1))


# speedup vs baseline: 24.6574x; 1.3131x over previous
import functools

import jax
import jax.numpy as jnp
from jax import lax
from jax.experimental import pallas as pl
from jax.experimental.pallas import tpu as pltpu
from jax.experimental.pallas import tpu_sc as plsc

N = 10000
E = 320000
IN_CH = 128
N_UNITS = 32
HEADS = 8
OUT_CH = 64

K = 128
EP = 331776
PER_TILE1 = EP // 16
NCH1 = PER_TILE1 // K
PER_TILE2 = EP // 32
NCH2 = PER_TILE2 // K

ACC1_R = 40064
DEN_R = 10112
ACC2_R = 10112

_MESH = plsc.VectorSubcoreMesh(core_axis_name="c", subcore_axis_name="s")


def _edge_weights(sa, da, wbuf):
    def body(e, c):
        s = sa[e, :] + da[e, :]
        wbuf[e, :] = jnp.exp(jnp.maximum(s, 0.2 * s))
        return c
    lax.fori_loop(0, K, body, 0, unroll=4)


def _l1_body(src_r, dst_r, as_r, ad_r, h1t_r, zacc_r, zden_r,
             acc_o, den_o,
             acc_sp, den_sp, sidx, didx, aoff, boff, hidx0, hidx1, hidx2,
             hidx3, sa, da, wbuf, hr0, hr1, hr2, hr3,
             sema, semb, semh0, semh1, semh2, semh3):
    core = lax.axis_index("c")
    sub = lax.axis_index("s")
    hidx = [hidx0, hidx1, hidx2, hidx3]
    hr = [hr0, hr1, hr2, hr3]
    semh = [semh0, semh1, semh2, semh3]

    pltpu.sync_copy(zacc_r, acc_sp.at[pl.ds(sub * 2504, 2504)])
    pltpu.sync_copy(zden_r, den_sp.at[pl.ds(sub * 632, 632)])
    plsc.subcore_barrier()

    def chunk(ci, c):
        base = sub * PER_TILE1 + ci * K
        pltpu.sync_copy(src_r.at[pl.ds(base, K)], sidx)
        pltpu.sync_copy(dst_r.at[pl.ds(base, K)], didx)
        toff = core * (N + 1)
        def aib(g, cc):
            s16 = sidx[pl.ds(g * 16, 16)]
            d16 = didx[pl.ds(g * 16, 16)]
            aoff[pl.ds(g * 16, 16)] = s16 + toff
            boff[pl.ds(g * 16, 16)] = d16 + toff
            s8 = s16 * 8 + core * 4
            hidx0[pl.ds(g * 16, 16)] = s8
            hidx1[pl.ds(g * 16, 16)] = s8 + 1
            hidx2[pl.ds(g * 16, 16)] = s8 + 2
            hidx3[pl.ds(g * 16, 16)] = s8 + 3
            return cc
        lax.fori_loop(0, K // 16, aib, 0, unroll=True)
        cpa = pltpu.async_copy(as_r.at[aoff], sa, sema)
        cpb = pltpu.async_copy(ad_r.at[boff], da, semb)
        cph = [pltpu.async_copy(h1t_r.at[hidx[j]], hr[j], semh[j])
               for j in range(4)]
        cpa.wait()
        cpb.wait()
        _edge_weights(sa, da, wbuf)
        pltpu.sync_copy(wbuf, den_sp.at[didx], add=True)
        for j in range(4):
            cph[j].wait()
            hrj = hr[j]
            def mul(e, cc):
                wv = jnp.full((16,), wbuf[e, :][j], jnp.float32)
                hrj[e, pl.ds(0, 16)] = hrj[e, pl.ds(0, 16)] * wv
                hrj[e, pl.ds(16, 16)] = hrj[e, pl.ds(16, 16)] * wv
                return cc
            lax.fori_loop(0, K, mul, 0, unroll=4)
            def sib(g, cc):
                boff[pl.ds(g * 16, 16)] = didx[pl.ds(g * 16, 16)] * 4 + j
                return cc
            lax.fori_loop(0, K // 16, sib, 0, unroll=True)
            pltpu.sync_copy(hr[j], acc_sp.at[boff], add=True)
        return c

    lax.fori_loop(0, NCH1, chunk, 0)
    plsc.subcore_barrier()

    pltpu.sync_copy(acc_sp.at[pl.ds(sub * 2504, 2504)],
                    acc_o.at[core, pl.ds(sub * 2504, 2504)])

    @pl.when(core == 0)
    def _():
        pltpu.sync_copy(den_sp.at[pl.ds(sub * 632, 632)],
                        den_o.at[pl.ds(sub * 632, 632)])


def _l2_body(src_r, dst_r, as_r, ad_r, h2t_r, zacc_r, zden_r,
             acc_o, den_o,
             acc_sp, den_sp, sidx, didx, sa, da, wbuf, hrows,
             sema, semb, semh):
    core = lax.axis_index("c")
    sub = lax.axis_index("s")

    pltpu.sync_copy(zacc_r, acc_sp.at[pl.ds(sub * 632, 632)])
    pltpu.sync_copy(zden_r, den_sp.at[pl.ds(sub * 632, 632)])
    plsc.subcore_barrier()

    def chunk(ci, c):
        base = core * (EP // 2) + sub * PER_TILE2 + ci * K
        pltpu.sync_copy(src_r.at[pl.ds(base, K)], sidx)
        pltpu.sync_copy(dst_r.at[pl.ds(base, K)], didx)
        cpa = pltpu.async_copy(as_r.at[sidx], sa, sema)
        cpb = pltpu.async_copy(ad_r.at[didx], da, semb)
        cph = pltpu.async_copy(h2t_r.at[sidx], hrows, semh)
        cpa.wait()
        cpb.wait()
        _edge_weights(sa, da, wbuf)
        pltpu.sync_copy(wbuf, den_sp.at[didx], add=True)
        cph.wait()
        def mul(e, cc):
            wv = jnp.full((16,), wbuf[e, :][0], jnp.float32)
            for q in range(4):
                hrows[e, pl.ds(16 * q, 16)] = hrows[e, pl.ds(16 * q, 16)] * wv
            return cc
        lax.fori_loop(0, K, mul, 0, unroll=4)
        pltpu.sync_copy(hrows, acc_sp.at[didx], add=True)
        return c

    lax.fori_loop(0, NCH2, chunk, 0)
    plsc.subcore_barrier()

    pltpu.sync_copy(acc_sp.at[pl.ds(sub * 632, 632)],
                    acc_o.at[core, pl.ds(sub * 632, 632)])
    pltpu.sync_copy(den_sp.at[pl.ds(sub * 632, 632)],
                    den_o.at[core, pl.ds(sub * 632, 632)])


_sc_l1 = pl.kernel(
    _l1_body,
    out_type=[
        jax.ShapeDtypeStruct((2, 40064, 32), jnp.float32),
        jax.ShapeDtypeStruct((10112, 16), jnp.float32),
    ],
    mesh=_MESH,
    compiler_params=pltpu.CompilerParams(use_tc_tiling_on_sc=False),
    scratch_types=[
        pltpu.VMEM_SHARED((ACC1_R, 32), jnp.float32),
        pltpu.VMEM_SHARED((DEN_R, 16), jnp.float32),
        pltpu.VMEM((K,), jnp.int32),
        pltpu.VMEM((K,), jnp.int32),
        pltpu.VMEM((K,), jnp.int32),
        pltpu.VMEM((K,), jnp.int32),
        pltpu.VMEM((K,), jnp.int32),
        pltpu.VMEM((K,), jnp.int32),
        pltpu.VMEM((K,), jnp.int32),
        pltpu.VMEM((K,), jnp.int32),
        pltpu.VMEM((K, 16), jnp.float32),
        pltpu.VMEM((K, 16), jnp.float32),
        pltpu.VMEM((K, 16), jnp.float32),
        pltpu.VMEM((K, 32), jnp.float32),
        pltpu.VMEM((K, 32), jnp.float32),
        pltpu.VMEM((K, 32), jnp.float32),
        pltpu.VMEM((K, 32), jnp.float32),
        pltpu.SemaphoreType.DMA,
        pltpu.SemaphoreType.DMA,
        pltpu.SemaphoreType.DMA,
        pltpu.SemaphoreType.DMA,
        pltpu.SemaphoreType.DMA,
        pltpu.SemaphoreType.DMA,
    ],
)

_sc_l2 = pl.kernel(
    _l2_body,
    out_type=[
        jax.ShapeDtypeStruct((2, 10112, 64), jnp.float32),
        jax.ShapeDtypeStruct((2, 10112, 16), jnp.float32),
    ],
    mesh=_MESH,
    compiler_params=pltpu.CompilerParams(use_tc_tiling_on_sc=False),
    scratch_types=[
        pltpu.VMEM_SHARED((ACC2_R, 64), jnp.float32),
        pltpu.VMEM_SHARED((DEN_R, 16), jnp.float32),
        pltpu.VMEM((K,), jnp.int32),
        pltpu.VMEM((K,), jnp.int32),
        pltpu.VMEM((K, 16), jnp.float32),
        pltpu.VMEM((K, 16), jnp.float32),
        pltpu.VMEM((K, 16), jnp.float32),
        pltpu.VMEM((K, 64), jnp.float32),
        pltpu.SemaphoreType.DMA,
        pltpu.SemaphoreType.DMA,
        pltpu.SemaphoreType.DMA,
    ],
)


_BLK = 1000


def _tc1_body(x_ref, w1_ref, asp_ref, adp_ref, h_ref, a_ref, b_ref):
    h = x_ref[...] @ w1_ref[...]
    h_ref[...] = h
    a_ref[...] = h @ asp_ref[...]
    b_ref[...] = h @ adp_ref[...]


_tc1 = pl.pallas_call(
    _tc1_body,
    grid=(N // _BLK,),
    in_specs=[
        pl.BlockSpec((_BLK, IN_CH), lambda i: (i, 0)),
        pl.BlockSpec((IN_CH, HEADS * N_UNITS), lambda i: (0, 0)),
        pl.BlockSpec((HEADS * N_UNITS, 16), lambda i: (0, 0)),
        pl.BlockSpec((HEADS * N_UNITS, 16), lambda i: (0, 0)),
    ],
    out_specs=[
        pl.BlockSpec((_BLK, HEADS * N_UNITS), lambda i: (i, 0)),
        pl.BlockSpec((_BLK, 16), lambda i: (i, 0)),
        pl.BlockSpec((_BLK, 16), lambda i: (i, 0)),
    ],
    out_shape=[
        jax.ShapeDtypeStruct((N, HEADS * N_UNITS), jnp.float32),
        jax.ShapeDtypeStruct((N, 16), jnp.float32),
        jax.ShapeDtypeStruct((N, 16), jnp.float32),
    ],
)


def _tc2_body(acc_ref, den_ref, e_ref, b1_ref, w2_ref, asp_ref, adp_ref,
              h2_ref, a_ref, b_ref):
    r = 1.0 / (den_ref[...] + 1e-16)
    rexp = r @ e_ref[...]
    h2p = jnp.maximum(acc_ref[...] * rexp + b1_ref[...], 0.0)
    h2 = h2p @ w2_ref[...]
    h2_ref[...] = h2
    a_ref[...] = h2 @ asp_ref[...]
    b_ref[...] = h2 @ adp_ref[...]


_tc2 = pl.pallas_call(
    _tc2_body,
    grid=(N // _BLK,),
    in_specs=[
        pl.BlockSpec((_BLK, HEADS * N_UNITS), lambda i: (i, 0)),
        pl.BlockSpec((_BLK, 16), lambda i: (i, 0)),
        pl.BlockSpec((16, HEADS * N_UNITS), lambda i: (0, 0)),
        pl.BlockSpec((1, HEADS * N_UNITS), lambda i: (0, 0)),
        pl.BlockSpec((HEADS * N_UNITS, OUT_CH), lambda i: (0, 0)),
        pl.BlockSpec((OUT_CH, 16), lambda i: (0, 0)),
        pl.BlockSpec((OUT_CH, 16), lambda i: (0, 0)),
    ],
    out_specs=[
        pl.BlockSpec((_BLK, OUT_CH), lambda i: (i, 0)),
        pl.BlockSpec((_BLK, 16), lambda i: (i, 0)),
        pl.BlockSpec((_BLK, 16), lambda i: (i, 0)),
    ],
    out_shape=[
        jax.ShapeDtypeStruct((N, OUT_CH), jnp.float32),
        jax.ShapeDtypeStruct((N, 16), jnp.float32),
        jax.ShapeDtypeStruct((N, 16), jnp.float32),
    ],
)


def _tc3_body(a0_ref, a1_ref, d0_ref, d1_ref, b2_ref, o_ref):
    den = d0_ref[...][:, 0:1] + d1_ref[...][:, 0:1] + 1e-16
    s = (a0_ref[...] + a1_ref[...]) / den + b2_ref[...]
    m = jnp.max(s, axis=-1, keepdims=True)
    ex = jnp.exp(s - m)
    o_ref[...] = (s - m) - jnp.log(jnp.sum(ex, axis=-1, keepdims=True))


_tc3 = pl.pallas_call(
    _tc3_body,
    grid=(N // _BLK,),
    in_specs=[
        pl.BlockSpec((_BLK, OUT_CH), lambda i: (i, 0)),
        pl.BlockSpec((_BLK, OUT_CH), lambda i: (i, 0)),
        pl.BlockSpec((_BLK, 16), lambda i: (i, 0)),
        pl.BlockSpec((_BLK, 16), lambda i: (i, 0)),
        pl.BlockSpec((1, OUT_CH), lambda i: (0, 0)),
    ],
    out_specs=pl.BlockSpec((_BLK, OUT_CH), lambda i: (i, 0)),
    out_shape=jax.ShapeDtypeStruct((N, OUT_CH), jnp.float32),
)


def kernel(x, edge_index, W1, att_src1, att_dst1, b1, W2, att_src2, att_dst2, b2):
    f32 = jnp.float32
    loop = jnp.arange(N, dtype=jnp.int32)
    padi = jnp.full((EP - E - N,), N, jnp.int32)
    src = jnp.concatenate([edge_index[0], loop, padi])
    dst = jnp.concatenate([edge_index[1], loop, padi])

    hd = jnp.arange(HEADS)
    AsP1 = jnp.zeros((HEADS, N_UNITS, 16), f32).at[hd, :, hd].set(att_src1)
    AsP1 = AsP1.reshape(HEADS * N_UNITS, 16)
    AdP1 = jnp.zeros((HEADS, N_UNITS, 16), f32).at[hd, :, hd].set(att_dst1)
    AdP1 = AdP1.reshape(HEADS * N_UNITS, 16)
    As2P = jnp.zeros((OUT_CH, 16), f32).at[:, 0].set(att_src2[0])
    Ad2P = jnp.zeros((OUT_CH, 16), f32).at[:, 0].set(att_dst2[0])
    E16 = jnp.concatenate(
        [jnp.kron(jnp.eye(HEADS, dtype=f32), jnp.ones((1, N_UNITS), f32)),
         jnp.zeros((8, HEADS * N_UNITS), f32)], axis=0)

    h1, aS1, aD1 = _tc1(x, W1, AsP1, AdP1)

    neg = jnp.full((1, 16), -1e30, f32)

    def _dbl(a, padrow):
        rot = jnp.concatenate([a[:, 4:8], a[:, 0:4], a[:, 8:16]], axis=1)
        return jnp.concatenate([a, padrow, rot, padrow], axis=0)

    aS1t = _dbl(aS1, neg)
    aD1t = _dbl(aD1, jnp.zeros((1, 16), f32))
    h1t = jnp.concatenate(
        [h1.reshape(N * HEADS, N_UNITS),
         jnp.zeros((HEADS, N_UNITS), f32)], axis=0)

    zacc1 = jnp.zeros((2504, 32), f32)
    zden = jnp.zeros((632, 16), f32)
    acc1, den1 = _sc_l1(src, dst, aS1t, aD1t, h1t, zacc1, zden)
    acc1 = acc1[:, :N * 4]
    den1 = den1[:N]

    acc1c = jnp.concatenate([acc1[0].reshape(N, 4 * N_UNITS),
                             acc1[1].reshape(N, 4 * N_UNITS)], axis=1)

    h2, aS2, aD2 = _tc2(acc1c, den1, E16, b1.reshape(1, -1), W2, As2P, Ad2P)

    aS2t = jnp.concatenate([aS2, neg], axis=0)
    aD2t = jnp.concatenate([aD2, jnp.zeros((1, 16), f32)], axis=0)
    h2t = jnp.concatenate([h2, jnp.zeros((1, OUT_CH), f32)], axis=0)

    zacc2 = jnp.zeros((632, 64), f32)
    acc2, den2 = _sc_l2(src, dst, aS2t, aD2t, h2t, zacc2, zden)
    acc2 = acc2[:, :N]
    den2 = den2[:, :N]

    return _tc3(acc2[0], acc2[1], den2[0], den2[1], b2.reshape(1, -1))


# speedup vs baseline: 25.0835x; 1.0173x over previous
import functools

import jax
import jax.numpy as jnp
from jax import lax
from jax.experimental import pallas as pl
from jax.experimental.pallas import tpu as pltpu
from jax.experimental.pallas import tpu_sc as plsc

N = 10000
E = 320000
IN_CH = 128
N_UNITS = 32
HEADS = 8
OUT_CH = 64

K = 128
EP = 331776
PER_TILE1 = EP // 16
NCH1 = PER_TILE1 // K
PER_TILE2 = EP // 32
NCH2 = PER_TILE2 // K

ACC1_R = 40064
DEN_R = 10112
ACC2_R = 10112

_MESH = plsc.VectorSubcoreMesh(core_axis_name="c", subcore_axis_name="s")


def _edge_weights(sa, da, wbuf):
    def body(e, c):
        s = sa[e, :] + da[e, :]
        wbuf[e, :] = jnp.exp(jnp.maximum(s, 0.2 * s))
        return c
    lax.fori_loop(0, K, body, 0, unroll=4)


def _l1_body(src_r, dst_r, as_r, ad_r, h1t_r, zacc_r, zden_r,
             acc_o, den_o,
             acc_sp, den_sp, sidx, didx, aoff, boff, hidx0, hidx1, hidx2,
             hidx3, sa, da, wbuf, hr0, hr1, hr2, hr3,
             sema, semb, semh0, semh1, semh2, semh3):
    core = lax.axis_index("c")
    sub = lax.axis_index("s")
    hidx = [hidx0, hidx1, hidx2, hidx3]
    hr = [hr0, hr1, hr2, hr3]
    semh = [semh0, semh1, semh2, semh3]

    pltpu.sync_copy(zacc_r, acc_sp.at[pl.ds(sub * 2504, 2504)])
    pltpu.sync_copy(zden_r, den_sp.at[pl.ds(sub * 632, 632)])
    plsc.subcore_barrier()

    def chunk(ci, c):
        base = sub * PER_TILE1 + ci * K
        pltpu.sync_copy(src_r.at[pl.ds(base, K)], sidx)
        pltpu.sync_copy(dst_r.at[pl.ds(base, K)], didx)
        toff = core * (N + 1)
        def aib(g, cc):
            s16 = sidx[pl.ds(g * 16, 16)]
            d16 = didx[pl.ds(g * 16, 16)]
            aoff[pl.ds(g * 16, 16)] = s16 + toff
            boff[pl.ds(g * 16, 16)] = d16 + toff
            s8 = s16 * 8 + core * 4
            hidx0[pl.ds(g * 16, 16)] = s8
            hidx1[pl.ds(g * 16, 16)] = s8 + 1
            hidx2[pl.ds(g * 16, 16)] = s8 + 2
            hidx3[pl.ds(g * 16, 16)] = s8 + 3
            return cc
        lax.fori_loop(0, K // 16, aib, 0, unroll=True)
        cpa = pltpu.async_copy(as_r.at[aoff], sa, sema)
        cpb = pltpu.async_copy(ad_r.at[boff], da, semb)
        cph = [pltpu.async_copy(h1t_r.at[hidx[j]], hr[j], semh[j])
               for j in range(4)]
        cpa.wait()
        cpb.wait()
        _edge_weights(sa, da, wbuf)
        pltpu.sync_copy(wbuf, den_sp.at[didx], add=True)
        for j in range(4):
            cph[j].wait()
        def mul(e, cc):
            wrow = wbuf[e, :]
            for j in range(4):
                wv = jnp.full((16,), wrow[j], jnp.float32)
                hrj = hr[j]
                hrj[e, pl.ds(0, 16)] = hrj[e, pl.ds(0, 16)] * wv
                hrj[e, pl.ds(16, 16)] = hrj[e, pl.ds(16, 16)] * wv
            return cc
        lax.fori_loop(0, K, mul, 0, unroll=4)
        for j in range(4):
            def sib(g, cc):
                boff[pl.ds(g * 16, 16)] = didx[pl.ds(g * 16, 16)] * 4 + j
                return cc
            lax.fori_loop(0, K // 16, sib, 0, unroll=True)
            pltpu.sync_copy(hr[j], acc_sp.at[boff], add=True)
        return c

    lax.fori_loop(0, NCH1, chunk, 0)
    plsc.subcore_barrier()

    pltpu.sync_copy(acc_sp.at[pl.ds(sub * 2504, 2504)],
                    acc_o.at[core, pl.ds(sub * 2504, 2504)])

    @pl.when(core == 0)
    def _():
        pltpu.sync_copy(den_sp.at[pl.ds(sub * 632, 632)],
                        den_o.at[pl.ds(sub * 632, 632)])


def _l2_body(src_r, dst_r, as_r, ad_r, h2t_r, zacc_r, zden_r,
             acc_o, den_o,
             acc_sp, den_sp, sidx, didx, sa, da, wbuf, hrows,
             sema, semb, semh):
    core = lax.axis_index("c")
    sub = lax.axis_index("s")

    pltpu.sync_copy(zacc_r, acc_sp.at[pl.ds(sub * 632, 632)])
    pltpu.sync_copy(zden_r, den_sp.at[pl.ds(sub * 632, 632)])
    plsc.subcore_barrier()

    def chunk(ci, c):
        base = core * (EP // 2) + sub * PER_TILE2 + ci * K
        pltpu.sync_copy(src_r.at[pl.ds(base, K)], sidx)
        pltpu.sync_copy(dst_r.at[pl.ds(base, K)], didx)
        cpa = pltpu.async_copy(as_r.at[sidx], sa, sema)
        cpb = pltpu.async_copy(ad_r.at[didx], da, semb)
        cph = pltpu.async_copy(h2t_r.at[sidx], hrows, semh)
        cpa.wait()
        cpb.wait()
        _edge_weights(sa, da, wbuf)
        pltpu.sync_copy(wbuf, den_sp.at[didx], add=True)
        cph.wait()
        def mul(e, cc):
            wv = jnp.full((16,), wbuf[e, :][0], jnp.float32)
            for q in range(4):
                hrows[e, pl.ds(16 * q, 16)] = hrows[e, pl.ds(16 * q, 16)] * wv
            return cc
        lax.fori_loop(0, K, mul, 0, unroll=4)
        pltpu.sync_copy(hrows, acc_sp.at[didx], add=True)
        return c

    lax.fori_loop(0, NCH2, chunk, 0)
    plsc.subcore_barrier()

    pltpu.sync_copy(acc_sp.at[pl.ds(sub * 632, 632)],
                    acc_o.at[core, pl.ds(sub * 632, 632)])
    pltpu.sync_copy(den_sp.at[pl.ds(sub * 632, 632)],
                    den_o.at[core, pl.ds(sub * 632, 632)])


_sc_l1 = pl.kernel(
    _l1_body,
    out_type=[
        jax.ShapeDtypeStruct((2, 40064, 32), jnp.float32),
        jax.ShapeDtypeStruct((10112, 16), jnp.float32),
    ],
    mesh=_MESH,
    compiler_params=pltpu.CompilerParams(use_tc_tiling_on_sc=False),
    scratch_types=[
        pltpu.VMEM_SHARED((ACC1_R, 32), jnp.float32),
        pltpu.VMEM_SHARED((DEN_R, 16), jnp.float32),
        pltpu.VMEM((K,), jnp.int32),
        pltpu.VMEM((K,), jnp.int32),
        pltpu.VMEM((K,), jnp.int32),
        pltpu.VMEM((K,), jnp.int32),
        pltpu.VMEM((K,), jnp.int32),
        pltpu.VMEM((K,), jnp.int32),
        pltpu.VMEM((K,), jnp.int32),
        pltpu.VMEM((K,), jnp.int32),
        pltpu.VMEM((K, 16), jnp.float32),
        pltpu.VMEM((K, 16), jnp.float32),
        pltpu.VMEM((K, 16), jnp.float32),
        pltpu.VMEM((K, 32), jnp.float32),
        pltpu.VMEM((K, 32), jnp.float32),
        pltpu.VMEM((K, 32), jnp.float32),
        pltpu.VMEM((K, 32), jnp.float32),
        pltpu.SemaphoreType.DMA,
        pltpu.SemaphoreType.DMA,
        pltpu.SemaphoreType.DMA,
        pltpu.SemaphoreType.DMA,
        pltpu.SemaphoreType.DMA,
        pltpu.SemaphoreType.DMA,
    ],
)

_sc_l2 = pl.kernel(
    _l2_body,
    out_type=[
        jax.ShapeDtypeStruct((2, 10112, 64), jnp.float32),
        jax.ShapeDtypeStruct((2, 10112, 16), jnp.float32),
    ],
    mesh=_MESH,
    compiler_params=pltpu.CompilerParams(use_tc_tiling_on_sc=False),
    scratch_types=[
        pltpu.VMEM_SHARED((ACC2_R, 64), jnp.float32),
        pltpu.VMEM_SHARED((DEN_R, 16), jnp.float32),
        pltpu.VMEM((K,), jnp.int32),
        pltpu.VMEM((K,), jnp.int32),
        pltpu.VMEM((K, 16), jnp.float32),
        pltpu.VMEM((K, 16), jnp.float32),
        pltpu.VMEM((K, 16), jnp.float32),
        pltpu.VMEM((K, 64), jnp.float32),
        pltpu.SemaphoreType.DMA,
        pltpu.SemaphoreType.DMA,
        pltpu.SemaphoreType.DMA,
    ],
)


_BLK = 1000


def _tc1_body(x_ref, w1_ref, asp_ref, adp_ref, h_ref, a_ref, b_ref):
    h = x_ref[...] @ w1_ref[...]
    h_ref[...] = h
    a_ref[...] = h @ asp_ref[...]
    b_ref[...] = h @ adp_ref[...]


_tc1 = pl.pallas_call(
    _tc1_body,
    grid=(N // _BLK,),
    in_specs=[
        pl.BlockSpec((_BLK, IN_CH), lambda i: (i, 0)),
        pl.BlockSpec((IN_CH, HEADS * N_UNITS), lambda i: (0, 0)),
        pl.BlockSpec((HEADS * N_UNITS, 16), lambda i: (0, 0)),
        pl.BlockSpec((HEADS * N_UNITS, 16), lambda i: (0, 0)),
    ],
    out_specs=[
        pl.BlockSpec((_BLK, HEADS * N_UNITS), lambda i: (i, 0)),
        pl.BlockSpec((_BLK, 16), lambda i: (i, 0)),
        pl.BlockSpec((_BLK, 16), lambda i: (i, 0)),
    ],
    out_shape=[
        jax.ShapeDtypeStruct((N, HEADS * N_UNITS), jnp.float32),
        jax.ShapeDtypeStruct((N, 16), jnp.float32),
        jax.ShapeDtypeStruct((N, 16), jnp.float32),
    ],
)


def _tc2_body(acc_ref, den_ref, e_ref, b1_ref, w2_ref, asp_ref, adp_ref,
              h2_ref, a_ref, b_ref):
    r = 1.0 / (den_ref[...] + 1e-16)
    rexp = r @ e_ref[...]
    h2p = jnp.maximum(acc_ref[...] * rexp + b1_ref[...], 0.0)
    h2 = h2p @ w2_ref[...]
    h2_ref[...] = h2
    a_ref[...] = h2 @ asp_ref[...]
    b_ref[...] = h2 @ adp_ref[...]


_tc2 = pl.pallas_call(
    _tc2_body,
    grid=(N // _BLK,),
    in_specs=[
        pl.BlockSpec((_BLK, HEADS * N_UNITS), lambda i: (i, 0)),
        pl.BlockSpec((_BLK, 16), lambda i: (i, 0)),
        pl.BlockSpec((16, HEADS * N_UNITS), lambda i: (0, 0)),
        pl.BlockSpec((1, HEADS * N_UNITS), lambda i: (0, 0)),
        pl.BlockSpec((HEADS * N_UNITS, OUT_CH), lambda i: (0, 0)),
        pl.BlockSpec((OUT_CH, 16), lambda i: (0, 0)),
        pl.BlockSpec((OUT_CH, 16), lambda i: (0, 0)),
    ],
    out_specs=[
        pl.BlockSpec((_BLK, OUT_CH), lambda i: (i, 0)),
        pl.BlockSpec((_BLK, 16), lambda i: (i, 0)),
        pl.BlockSpec((_BLK, 16), lambda i: (i, 0)),
    ],
    out_shape=[
        jax.ShapeDtypeStruct((N, OUT_CH), jnp.float32),
        jax.ShapeDtypeStruct((N, 16), jnp.float32),
        jax.ShapeDtypeStruct((N, 16), jnp.float32),
    ],
)


def _tc3_body(a0_ref, a1_ref, d0_ref, d1_ref, b2_ref, o_ref):
    den = d0_ref[...][:, 0:1] + d1_ref[...][:, 0:1] + 1e-16
    s = (a0_ref[...] + a1_ref[...]) / den + b2_ref[...]
    m = jnp.max(s, axis=-1, keepdims=True)
    ex = jnp.exp(s - m)
    o_ref[...] = (s - m) - jnp.log(jnp.sum(ex, axis=-1, keepdims=True))


_tc3 = pl.pallas_call(
    _tc3_body,
    grid=(N // _BLK,),
    in_specs=[
        pl.BlockSpec((_BLK, OUT_CH), lambda i: (i, 0)),
        pl.BlockSpec((_BLK, OUT_CH), lambda i: (i, 0)),
        pl.BlockSpec((_BLK, 16), lambda i: (i, 0)),
        pl.BlockSpec((_BLK, 16), lambda i: (i, 0)),
        pl.BlockSpec((1, OUT_CH), lambda i: (0, 0)),
    ],
    out_specs=pl.BlockSpec((_BLK, OUT_CH), lambda i: (i, 0)),
    out_shape=jax.ShapeDtypeStruct((N, OUT_CH), jnp.float32),
)


def kernel(x, edge_index, W1, att_src1, att_dst1, b1, W2, att_src2, att_dst2, b2):
    f32 = jnp.float32
    loop = jnp.arange(N, dtype=jnp.int32)
    padi = jnp.full((EP - E - N,), N, jnp.int32)
    src = jnp.concatenate([edge_index[0], loop, padi])
    dst = jnp.concatenate([edge_index[1], loop, padi])

    hd = jnp.arange(HEADS)
    AsP1 = jnp.zeros((HEADS, N_UNITS, 16), f32).at[hd, :, hd].set(att_src1)
    AsP1 = AsP1.reshape(HEADS * N_UNITS, 16)
    AdP1 = jnp.zeros((HEADS, N_UNITS, 16), f32).at[hd, :, hd].set(att_dst1)
    AdP1 = AdP1.reshape(HEADS * N_UNITS, 16)
    As2P = jnp.zeros((OUT_CH, 16), f32).at[:, 0].set(att_src2[0])
    Ad2P = jnp.zeros((OUT_CH, 16), f32).at[:, 0].set(att_dst2[0])
    E16 = jnp.concatenate(
        [jnp.kron(jnp.eye(HEADS, dtype=f32), jnp.ones((1, N_UNITS), f32)),
         jnp.zeros((8, HEADS * N_UNITS), f32)], axis=0)

    h1, aS1, aD1 = _tc1(x, W1, AsP1, AdP1)

    neg = jnp.full((1, 16), -1e30, f32)

    def _dbl(a, padrow):
        rot = jnp.concatenate([a[:, 4:8], a[:, 0:4], a[:, 8:16]], axis=1)
        return jnp.concatenate([a, padrow, rot, padrow], axis=0)

    aS1t = _dbl(aS1, neg)
    aD1t = _dbl(aD1, jnp.zeros((1, 16), f32))
    h1t = jnp.concatenate(
        [h1.reshape(N * HEADS, N_UNITS),
         jnp.zeros((HEADS, N_UNITS), f32)], axis=0)

    zacc1 = jnp.zeros((2504, 32), f32)
    zden = jnp.zeros((632, 16), f32)
    acc1, den1 = _sc_l1(src, dst, aS1t, aD1t, h1t, zacc1, zden)
    acc1 = acc1[:, :N * 4]
    den1 = den1[:N]

    acc1c = jnp.concatenate([acc1[0].reshape(N, 4 * N_UNITS),
                             acc1[1].reshape(N, 4 * N_UNITS)], axis=1)

    h2, aS2, aD2 = _tc2(acc1c, den1, E16, b1.reshape(1, -1), W2, As2P, Ad2P)

    aS2t = jnp.concatenate([aS2, neg], axis=0)
    aD2t = jnp.concatenate([aD2, jnp.zeros((1, 16), f32)], axis=0)
    h2t = jnp.concatenate([h2, jnp.zeros((1, OUT_CH), f32)], axis=0)

    zacc2 = jnp.zeros((632, 64), f32)
    acc2, den2 = _sc_l2(src, dst, aS2t, aD2t, h2t, zacc2, zden)
    acc2 = acc2[:, :N]
    den2 = den2[:, :N]

    return _tc3(acc2[0], acc2[1], den2[0], den2[1], b2.reshape(1, -1))


# speedup vs baseline: 28.6138x; 1.1407x over previous
import functools

import jax
import jax.numpy as jnp
from jax import lax
from jax.experimental import pallas as pl
from jax.experimental.pallas import tpu as pltpu
from jax.experimental.pallas import tpu_sc as plsc

N = 10000
E = 320000
IN_CH = 128
N_UNITS = 32
HEADS = 8
OUT_CH = 64

K = 96
EP = 331776
PER_TILE1 = EP // 16
NCH1 = PER_TILE1 // K
PER_TILE2 = EP // 32
NCH2 = PER_TILE2 // K

ACC1_R = 40064
DEN_R = 10112
ACC2_R = 10112

_MESH = plsc.VectorSubcoreMesh(core_axis_name="c", subcore_axis_name="s")


def _edge_weights(sa, da, wbuf):
    def body(e, c):
        s = sa[e, :] + da[e, :]
        wbuf[e, :] = jnp.exp(jnp.maximum(s, 0.2 * s))
        return c
    lax.fori_loop(0, K, body, 0, unroll=4)


def _l1_body(epk_r, as_r, ad_r, h1t_r, zacc_r, zden_r,
             acc_o, den_o,
             acc_sp, den_sp, sidx_all, didx_all, *rest):
    core = lax.axis_index("c")
    sub = lax.axis_index("s")
    bufs = [rest[0:14], rest[14:28]]
    sems = [rest[28:34], rest[34:40]]
    sidx2 = (sidx_all, didx_all)

    pltpu.sync_copy(zacc_r, acc_sp.at[pl.ds(sub * 2504, 632)])
    pltpu.sync_copy(zacc_r, acc_sp.at[pl.ds(sub * 2504 + 632, 632)])
    pltpu.sync_copy(zacc_r, acc_sp.at[pl.ds(sub * 2504 + 1264, 632)])
    pltpu.sync_copy(zacc_r.at[pl.ds(0, 608)],
                    acc_sp.at[pl.ds(sub * 2504 + 1896, 608)])
    pltpu.sync_copy(zden_r, den_sp.at[pl.ds(sub * 632, 632)])
    plsc.subcore_barrier()

    toff = core * (N + 1)

    def issue(ci, p):
        aoff, boff, h0, h1, h2, h3, didx, sa, da, wbuf = bufs[p][:10]
        sema, semb = sems[p][:2]
        semh = sems[p][2:6]
        hr = bufs[p][10:14]
        sidx = sidx2[p]
        base = sub * PER_TILE1 + ci * K
        pltpu.sync_copy(epk_r.at[pl.ds(base, K)], sidx)

        def aib(g, cc):
            p16 = sidx[pl.ds(g * 16, 16)]
            s16 = jnp.right_shift(p16, 14)
            d16 = jnp.bitwise_and(p16, 16383)
            didx[pl.ds(g * 16, 16)] = d16
            aoff[pl.ds(g * 16, 16)] = s16 + toff
            boff[pl.ds(g * 16, 16)] = d16 + toff
            s8 = s16 * 8 + core * 4
            h0[pl.ds(g * 16, 16)] = s8
            h1[pl.ds(g * 16, 16)] = s8 + 1
            h2[pl.ds(g * 16, 16)] = s8 + 2
            h3[pl.ds(g * 16, 16)] = s8 + 3
            return cc
        lax.fori_loop(0, K // 16, aib, 0, unroll=True)
        pltpu.async_copy(as_r.at[aoff], sa, sema)
        pltpu.async_copy(ad_r.at[boff], da, semb)
        hidx = (h0, h1, h2, h3)
        for j in range(4):
            pltpu.async_copy(h1t_r.at[hidx[j]], hr[j], semh[j])

    def process(p):
        aoff, boff, h0, h1, h2, h3, didx, sa, da, wbuf = bufs[p][:10]
        sema, semb = sems[p][:2]
        semh = sems[p][2:6]
        hr = bufs[p][10:14]
        hidx = (h0, h1, h2, h3)
        pltpu.make_async_copy(as_r.at[aoff], sa, sema).wait()
        pltpu.make_async_copy(ad_r.at[boff], da, semb).wait()
        _edge_weights(sa, da, wbuf)
        pltpu.sync_copy(wbuf, den_sp.at[didx], add=True)
        for j in range(4):
            pltpu.make_async_copy(h1t_r.at[hidx[j]], hr[j], semh[j]).wait()

        def mul(e, cc):
            wrow = wbuf[e, :]
            for j in range(4):
                wv = jnp.full((16,), wrow[j], jnp.float32)
                hrj = hr[j]
                hrj[e, pl.ds(0, 16)] = hrj[e, pl.ds(0, 16)] * wv
                hrj[e, pl.ds(16, 16)] = hrj[e, pl.ds(16, 16)] * wv
            return cc
        lax.fori_loop(0, K, mul, 0, unroll=4)
        for j in range(4):
            def sib(g, cc):
                boff[pl.ds(g * 16, 16)] = didx[pl.ds(g * 16, 16)] * 4 + j
                return cc
            lax.fori_loop(0, K // 16, sib, 0, unroll=True)
            pltpu.sync_copy(hr[j], acc_sp.at[boff], add=True)

    issue(0, 0)

    def body(i, c):
        issue(2 * i + 1, 1)
        process(0)
        issue(2 * i + 2, 0)
        process(1)
        return c

    lax.fori_loop(0, NCH1 // 2 - 1, body, 0)
    issue(NCH1 - 1, 1)
    process(0)
    process(1)
    plsc.subcore_barrier()

    pltpu.sync_copy(acc_sp.at[pl.ds(sub * 2504, 2504)],
                    acc_o.at[core, pl.ds(sub * 2504, 2504)])

    @pl.when(core == 0)
    def _():
        pltpu.sync_copy(den_sp.at[pl.ds(sub * 632, 632)],
                        den_o.at[pl.ds(sub * 632, 632)])


def _l2_body(epk_r, as_r, ad_r, h2t_r, zacc_r, zden_r,
             acc_o, den_o,
             acc_sp, den_sp, pidx, sidx, didx, sa, da, wbuf, hrows,
             sema, semb, semh):
    core = lax.axis_index("c")
    sub = lax.axis_index("s")

    pltpu.sync_copy(zacc_r, acc_sp.at[pl.ds(sub * 632, 632)])
    pltpu.sync_copy(zden_r, den_sp.at[pl.ds(sub * 632, 632)])
    plsc.subcore_barrier()

    def chunk(ci, c):
        base = core * (EP // 2) + sub * PER_TILE2 + ci * K
        pltpu.sync_copy(epk_r.at[pl.ds(base, K)], pidx)

        def ub(g, cc):
            p16 = pidx[pl.ds(g * 16, 16)]
            sidx[pl.ds(g * 16, 16)] = jnp.right_shift(p16, 14)
            didx[pl.ds(g * 16, 16)] = jnp.bitwise_and(p16, 16383)
            return cc
        lax.fori_loop(0, K // 16, ub, 0, unroll=True)
        cpa = pltpu.async_copy(as_r.at[sidx], sa, sema)
        cpb = pltpu.async_copy(ad_r.at[didx], da, semb)
        cph = pltpu.async_copy(h2t_r.at[sidx], hrows, semh)
        cpa.wait()
        cpb.wait()
        _edge_weights(sa, da, wbuf)
        pltpu.sync_copy(wbuf, den_sp.at[didx], add=True)
        cph.wait()
        def mul(e, cc):
            wv = jnp.full((16,), wbuf[e, :][0], jnp.float32)
            for q in range(4):
                hrows[e, pl.ds(16 * q, 16)] = hrows[e, pl.ds(16 * q, 16)] * wv
            return cc
        lax.fori_loop(0, K, mul, 0, unroll=4)
        pltpu.sync_copy(hrows, acc_sp.at[didx], add=True)
        return c

    lax.fori_loop(0, NCH2, chunk, 0)
    plsc.subcore_barrier()

    pltpu.sync_copy(acc_sp.at[pl.ds(sub * 632, 632)],
                    acc_o.at[core, pl.ds(sub * 632, 632)])
    pltpu.sync_copy(den_sp.at[pl.ds(sub * 632, 632)],
                    den_o.at[core, pl.ds(sub * 632, 632)])


_sc_l1 = pl.kernel(
    _l1_body,
    out_type=[
        jax.ShapeDtypeStruct((2, 40064, 32), jnp.float32),
        jax.ShapeDtypeStruct((10112, 16), jnp.float32),
    ],
    mesh=_MESH,
    compiler_params=pltpu.CompilerParams(use_tc_tiling_on_sc=False),
    scratch_types=[
        pltpu.VMEM_SHARED((ACC1_R, 32), jnp.float32),
        pltpu.VMEM_SHARED((DEN_R, 16), jnp.float32),
        pltpu.VMEM((K,), jnp.int32),
        pltpu.VMEM((K,), jnp.int32),
        pltpu.VMEM((K,), jnp.int32),
        pltpu.VMEM((K,), jnp.int32),
        pltpu.VMEM((K,), jnp.int32),
        pltpu.VMEM((K,), jnp.int32),
        pltpu.VMEM((K,), jnp.int32),
        pltpu.VMEM((K,), jnp.int32),
        pltpu.VMEM((K,), jnp.int32),
        pltpu.VMEM((K, 16), jnp.float32),
        pltpu.VMEM((K, 16), jnp.float32),
        pltpu.VMEM((K, 16), jnp.float32),
        pltpu.VMEM((K, 32), jnp.float32),
        pltpu.VMEM((K, 32), jnp.float32),
        pltpu.VMEM((K, 32), jnp.float32),
        pltpu.VMEM((K, 32), jnp.float32),
        pltpu.VMEM((K,), jnp.int32),
        pltpu.VMEM((K,), jnp.int32),
        pltpu.VMEM((K,), jnp.int32),
        pltpu.VMEM((K,), jnp.int32),
        pltpu.VMEM((K,), jnp.int32),
        pltpu.VMEM((K,), jnp.int32),
        pltpu.VMEM((K,), jnp.int32),
        pltpu.VMEM((K, 16), jnp.float32),
        pltpu.VMEM((K, 16), jnp.float32),
        pltpu.VMEM((K, 16), jnp.float32),
        pltpu.VMEM((K, 32), jnp.float32),
        pltpu.VMEM((K, 32), jnp.float32),
        pltpu.VMEM((K, 32), jnp.float32),
        pltpu.VMEM((K, 32), jnp.float32),
        pltpu.SemaphoreType.DMA,
        pltpu.SemaphoreType.DMA,
        pltpu.SemaphoreType.DMA,
        pltpu.SemaphoreType.DMA,
        pltpu.SemaphoreType.DMA,
        pltpu.SemaphoreType.DMA,
        pltpu.SemaphoreType.DMA,
        pltpu.SemaphoreType.DMA,
        pltpu.SemaphoreType.DMA,
        pltpu.SemaphoreType.DMA,
        pltpu.SemaphoreType.DMA,
        pltpu.SemaphoreType.DMA,
    ],
)

_sc_l2 = pl.kernel(
    _l2_body,
    out_type=[
        jax.ShapeDtypeStruct((2, 10112, 64), jnp.float32),
        jax.ShapeDtypeStruct((2, 10112, 16), jnp.float32),
    ],
    mesh=_MESH,
    compiler_params=pltpu.CompilerParams(use_tc_tiling_on_sc=False),
    scratch_types=[
        pltpu.VMEM_SHARED((ACC2_R, 64), jnp.float32),
        pltpu.VMEM_SHARED((DEN_R, 16), jnp.float32),
        pltpu.VMEM((K,), jnp.int32),
        pltpu.VMEM((K,), jnp.int32),
        pltpu.VMEM((K,), jnp.int32),
        pltpu.VMEM((K, 16), jnp.float32),
        pltpu.VMEM((K, 16), jnp.float32),
        pltpu.VMEM((K, 16), jnp.float32),
        pltpu.VMEM((K, 64), jnp.float32),
        pltpu.SemaphoreType.DMA,
        pltpu.SemaphoreType.DMA,
        pltpu.SemaphoreType.DMA,
    ],
)


_BLK = 1000


def _tc1_body(x_ref, w1_ref, asp_ref, adp_ref, h_ref, a_ref, b_ref):
    h = x_ref[...] @ w1_ref[...]
    h_ref[...] = h
    a_ref[...] = h @ asp_ref[...]
    b_ref[...] = h @ adp_ref[...]


_tc1 = pl.pallas_call(
    _tc1_body,
    grid=(N // _BLK,),
    in_specs=[
        pl.BlockSpec((_BLK, IN_CH), lambda i: (i, 0)),
        pl.BlockSpec((IN_CH, HEADS * N_UNITS), lambda i: (0, 0)),
        pl.BlockSpec((HEADS * N_UNITS, 16), lambda i: (0, 0)),
        pl.BlockSpec((HEADS * N_UNITS, 16), lambda i: (0, 0)),
    ],
    out_specs=[
        pl.BlockSpec((_BLK, HEADS * N_UNITS), lambda i: (i, 0)),
        pl.BlockSpec((_BLK, 16), lambda i: (i, 0)),
        pl.BlockSpec((_BLK, 16), lambda i: (i, 0)),
    ],
    out_shape=[
        jax.ShapeDtypeStruct((N, HEADS * N_UNITS), jnp.float32),
        jax.ShapeDtypeStruct((N, 16), jnp.float32),
        jax.ShapeDtypeStruct((N, 16), jnp.float32),
    ],
)


def _tc2_body(acc_ref, den_ref, e_ref, b1_ref, w2_ref, asp_ref, adp_ref,
              h2_ref, a_ref, b_ref):
    r = 1.0 / (den_ref[...] + 1e-16)
    rexp = r @ e_ref[...]
    h2p = jnp.maximum(acc_ref[...] * rexp + b1_ref[...], 0.0)
    h2 = h2p @ w2_ref[...]
    h2_ref[...] = h2
    a_ref[...] = h2 @ asp_ref[...]
    b_ref[...] = h2 @ adp_ref[...]


_tc2 = pl.pallas_call(
    _tc2_body,
    grid=(N // _BLK,),
    in_specs=[
        pl.BlockSpec((_BLK, HEADS * N_UNITS), lambda i: (i, 0)),
        pl.BlockSpec((_BLK, 16), lambda i: (i, 0)),
        pl.BlockSpec((16, HEADS * N_UNITS), lambda i: (0, 0)),
        pl.BlockSpec((1, HEADS * N_UNITS), lambda i: (0, 0)),
        pl.BlockSpec((HEADS * N_UNITS, OUT_CH), lambda i: (0, 0)),
        pl.BlockSpec((OUT_CH, 16), lambda i: (0, 0)),
        pl.BlockSpec((OUT_CH, 16), lambda i: (0, 0)),
    ],
    out_specs=[
        pl.BlockSpec((_BLK, OUT_CH), lambda i: (i, 0)),
        pl.BlockSpec((_BLK, 16), lambda i: (i, 0)),
        pl.BlockSpec((_BLK, 16), lambda i: (i, 0)),
    ],
    out_shape=[
        jax.ShapeDtypeStruct((N, OUT_CH), jnp.float32),
        jax.ShapeDtypeStruct((N, 16), jnp.float32),
        jax.ShapeDtypeStruct((N, 16), jnp.float32),
    ],
)


def _tc3_body(a0_ref, a1_ref, d0_ref, d1_ref, b2_ref, o_ref):
    den = d0_ref[...][:, 0:1] + d1_ref[...][:, 0:1] + 1e-16
    s = (a0_ref[...] + a1_ref[...]) / den + b2_ref[...]
    m = jnp.max(s, axis=-1, keepdims=True)
    ex = jnp.exp(s - m)
    o_ref[...] = (s - m) - jnp.log(jnp.sum(ex, axis=-1, keepdims=True))


_tc3 = pl.pallas_call(
    _tc3_body,
    grid=(N // _BLK,),
    in_specs=[
        pl.BlockSpec((_BLK, OUT_CH), lambda i: (i, 0)),
        pl.BlockSpec((_BLK, OUT_CH), lambda i: (i, 0)),
        pl.BlockSpec((_BLK, 16), lambda i: (i, 0)),
        pl.BlockSpec((_BLK, 16), lambda i: (i, 0)),
        pl.BlockSpec((1, OUT_CH), lambda i: (0, 0)),
    ],
    out_specs=pl.BlockSpec((_BLK, OUT_CH), lambda i: (i, 0)),
    out_shape=jax.ShapeDtypeStruct((N, OUT_CH), jnp.float32),
)


def kernel(x, edge_index, W1, att_src1, att_dst1, b1, W2, att_src2, att_dst2, b2):
    f32 = jnp.float32
    loop = jnp.arange(N, dtype=jnp.int32)
    padi = jnp.full((EP - E - N,), N, jnp.int32)
    src = jnp.concatenate([edge_index[0], loop, padi])
    dst = jnp.concatenate([edge_index[1], loop, padi])
    epk = src * 16384 + dst

    hd = jnp.arange(HEADS)
    AsP1 = jnp.zeros((HEADS, N_UNITS, 16), f32).at[hd, :, hd].set(att_src1)
    AsP1 = AsP1.reshape(HEADS * N_UNITS, 16)
    AdP1 = jnp.zeros((HEADS, N_UNITS, 16), f32).at[hd, :, hd].set(att_dst1)
    AdP1 = AdP1.reshape(HEADS * N_UNITS, 16)
    As2P = jnp.zeros((OUT_CH, 16), f32).at[:, 0].set(att_src2[0])
    Ad2P = jnp.zeros((OUT_CH, 16), f32).at[:, 0].set(att_dst2[0])
    E16 = jnp.concatenate(
        [jnp.kron(jnp.eye(HEADS, dtype=f32), jnp.ones((1, N_UNITS), f32)),
         jnp.zeros((8, HEADS * N_UNITS), f32)], axis=0)

    h1, aS1, aD1 = _tc1(x, W1, AsP1, AdP1)

    neg = jnp.full((1, 16), -1e30, f32)

    def _dbl(a, padrow):
        rot = jnp.concatenate([a[:, 4:8], a[:, 0:4], a[:, 8:16]], axis=1)
        return jnp.concatenate([a, padrow, rot, padrow], axis=0)

    aS1t = _dbl(aS1, neg)
    aD1t = _dbl(aD1, jnp.zeros((1, 16), f32))
    h1t = jnp.concatenate(
        [h1.reshape(N * HEADS, N_UNITS),
         jnp.zeros((HEADS, N_UNITS), f32)], axis=0)

    zacc1 = jnp.zeros((632, 32), f32)
    zden = jnp.zeros((632, 16), f32)
    acc1, den1 = _sc_l1(epk, aS1t, aD1t, h1t, zacc1, zden)
    acc1 = acc1[:, :N * 4]
    den1 = den1[:N]

    acc1c = jnp.concatenate([acc1[0].reshape(N, 4 * N_UNITS),
                             acc1[1].reshape(N, 4 * N_UNITS)], axis=1)

    h2, aS2, aD2 = _tc2(acc1c, den1, E16, b1.reshape(1, -1), W2, As2P, Ad2P)

    aS2t = jnp.concatenate([aS2, neg], axis=0)
    aD2t = jnp.concatenate([aD2, jnp.zeros((1, 16), f32)], axis=0)
    h2t = jnp.concatenate([h2, jnp.zeros((1, OUT_CH), f32)], axis=0)

    zacc2 = jnp.zeros((632, 64), f32)
    acc2, den2 = _sc_l2(epk, aS2t, aD2t, h2t, zacc2, zden)
    acc2 = acc2[:, :N]
    den2 = den2[:, :N]

    return _tc3(acc2[0], acc2[1], den2[0], den2[1], b2.reshape(1, -1))


# speedup vs baseline: 30.3839x; 1.0619x over previous
import functools

import jax
import jax.numpy as jnp
from jax import lax
from jax.experimental import pallas as pl
from jax.experimental.pallas import tpu as pltpu
from jax.experimental.pallas import tpu_sc as plsc

N = 10000
E = 320000
IN_CH = 128
N_UNITS = 32
HEADS = 8
OUT_CH = 64

K = 96
EP = 331776
PER_TILE1 = EP // 16
NCH1 = PER_TILE1 // K
PER_TILE2 = EP // 32
NCH2 = PER_TILE2 // K

ACC1_R = 40064
DEN_R = 10112
ACC2_R = 10112

_MESH = plsc.VectorSubcoreMesh(core_axis_name="c", subcore_axis_name="s")


def _edge_weights(sa, da, wbuf):
    def body(e, c):
        s = sa[e, :] + da[e, :]
        wbuf[e, :] = jnp.exp(jnp.maximum(s, 0.2 * s))
        return c
    lax.fori_loop(0, K, body, 0, unroll=4)


def _l1_body(epk_r, as_r, ad_r, h1t_r, zacc_r, zden_r,
             acc_o, den_o,
             acc_sp, den_sp, sidx_all, didx_all, *rest):
    core = lax.axis_index("c")
    sub = lax.axis_index("s")
    bufs = [rest[0:14], rest[14:28]]
    sems = [rest[28:34], rest[34:40]]
    sidx2 = (sidx_all, didx_all)

    pltpu.sync_copy(zacc_r, acc_sp.at[pl.ds(sub * 2504, 632)])
    pltpu.sync_copy(zacc_r, acc_sp.at[pl.ds(sub * 2504 + 632, 632)])
    pltpu.sync_copy(zacc_r, acc_sp.at[pl.ds(sub * 2504 + 1264, 632)])
    pltpu.sync_copy(zacc_r.at[pl.ds(0, 608)],
                    acc_sp.at[pl.ds(sub * 2504 + 1896, 608)])
    pltpu.sync_copy(zden_r, den_sp.at[pl.ds(sub * 632, 632)])
    plsc.subcore_barrier()

    toff = core * (N + 1)

    def issue(ci, p):
        aoff, boff, h0, h1, h2, h3, didx, sa, da, wbuf = bufs[p][:10]
        sema, semb = sems[p][:2]
        semh = sems[p][2:6]
        hr = bufs[p][10:14]
        sidx = sidx2[p]
        base = sub * PER_TILE1 + ci * K
        pltpu.sync_copy(epk_r.at[pl.ds(base, K)], sidx)

        def aib(g, cc):
            p16 = sidx[pl.ds(g * 16, 16)]
            s16 = jnp.right_shift(p16, 14)
            d16 = jnp.bitwise_and(p16, 16383)
            didx[pl.ds(g * 16, 16)] = d16
            aoff[pl.ds(g * 16, 16)] = s16 + toff
            boff[pl.ds(g * 16, 16)] = d16 + toff
            s8 = s16 * 8 + core * 4
            h0[pl.ds(g * 16, 16)] = s8
            h1[pl.ds(g * 16, 16)] = s8 + 1
            h2[pl.ds(g * 16, 16)] = s8 + 2
            h3[pl.ds(g * 16, 16)] = s8 + 3
            return cc
        lax.fori_loop(0, K // 16, aib, 0, unroll=True)
        pltpu.async_copy(as_r.at[aoff], sa, sema)
        pltpu.async_copy(ad_r.at[boff], da, semb)
        hidx = (h0, h1, h2, h3)
        for j in range(4):
            pltpu.async_copy(h1t_r.at[hidx[j]], hr[j], semh[j])

    def process(p):
        aoff, boff, h0, h1, h2, h3, didx, sa, da, wbuf = bufs[p][:10]
        sema, semb = sems[p][:2]
        semh = sems[p][2:6]
        hr = bufs[p][10:14]
        hidx = (h0, h1, h2, h3)
        pltpu.make_async_copy(as_r.at[aoff], sa, sema).wait()
        pltpu.make_async_copy(ad_r.at[boff], da, semb).wait()
        _edge_weights(sa, da, wbuf)
        pltpu.sync_copy(wbuf, den_sp.at[didx], add=True)
        for j in range(4):
            pltpu.make_async_copy(h1t_r.at[hidx[j]], hr[j], semh[j]).wait()

        def mul(e, cc):
            wrow = wbuf[e, :]
            for j in range(4):
                wv = jnp.full((16,), wrow[j], jnp.float32)
                hrj = hr[j]
                hrj[e, pl.ds(0, 16)] = hrj[e, pl.ds(0, 16)] * wv
                hrj[e, pl.ds(16, 16)] = hrj[e, pl.ds(16, 16)] * wv
            return cc
        lax.fori_loop(0, K, mul, 0, unroll=4)
        for j in range(4):
            def sib(g, cc):
                boff[pl.ds(g * 16, 16)] = didx[pl.ds(g * 16, 16)] * 4 + j
                return cc
            lax.fori_loop(0, K // 16, sib, 0, unroll=True)
            pltpu.sync_copy(hr[j], acc_sp.at[boff], add=True)

    issue(0, 0)

    def body(i, c):
        issue(2 * i + 1, 1)
        process(0)
        issue(2 * i + 2, 0)
        process(1)
        return c

    lax.fori_loop(0, NCH1 // 2 - 1, body, 0)
    issue(NCH1 - 1, 1)
    process(0)
    process(1)
    plsc.subcore_barrier()

    pltpu.sync_copy(acc_sp.at[pl.ds(sub * 2504, 2504)],
                    acc_o.at[core, pl.ds(sub * 2504, 2504)])

    @pl.when(core == 0)
    def _():
        pltpu.sync_copy(den_sp.at[pl.ds(sub * 632, 632)],
                        den_o.at[pl.ds(sub * 632, 632)])


def _l2_body(epk_r, as_r, ad_r, h2t_r, zacc_r, zden_r,
             acc_o, den_o,
             acc_sp, den_sp, *rest):
    core = lax.axis_index("c")
    sub = lax.axis_index("s")
    bufs = [rest[0:7], rest[7:14]]
    sems = [rest[14:17], rest[17:20]]

    pltpu.sync_copy(zacc_r, acc_sp.at[pl.ds(sub * 632, 632)])
    pltpu.sync_copy(zden_r, den_sp.at[pl.ds(sub * 632, 632)])
    plsc.subcore_barrier()

    def issue(ci, p):
        pidx, sidx, didx, sa, da, wbuf, hrows = bufs[p]
        sema, semb, semh = sems[p]
        base = core * (EP // 2) + sub * PER_TILE2 + ci * K
        pltpu.sync_copy(epk_r.at[pl.ds(base, K)], pidx)

        def ub(g, cc):
            p16 = pidx[pl.ds(g * 16, 16)]
            sidx[pl.ds(g * 16, 16)] = jnp.right_shift(p16, 14)
            didx[pl.ds(g * 16, 16)] = jnp.bitwise_and(p16, 16383)
            return cc
        lax.fori_loop(0, K // 16, ub, 0, unroll=True)
        pltpu.async_copy(as_r.at[sidx], sa, sema)
        pltpu.async_copy(ad_r.at[didx], da, semb)
        pltpu.async_copy(h2t_r.at[sidx], hrows, semh)

    def process(p):
        pidx, sidx, didx, sa, da, wbuf, hrows = bufs[p]
        sema, semb, semh = sems[p]
        pltpu.make_async_copy(as_r.at[sidx], sa, sema).wait()
        pltpu.make_async_copy(ad_r.at[didx], da, semb).wait()
        _edge_weights(sa, da, wbuf)
        pltpu.sync_copy(wbuf, den_sp.at[didx], add=True)
        pltpu.make_async_copy(h2t_r.at[sidx], hrows, semh).wait()

        def mul(e, cc):
            wv = jnp.full((16,), wbuf[e, :][0], jnp.float32)
            for q in range(4):
                hrows[e, pl.ds(16 * q, 16)] = hrows[e, pl.ds(16 * q, 16)] * wv
            return cc
        lax.fori_loop(0, K, mul, 0, unroll=4)
        pltpu.sync_copy(hrows, acc_sp.at[didx], add=True)

    issue(0, 0)

    def body(i, c):
        issue(2 * i + 1, 1)
        process(0)
        issue(2 * i + 2, 0)
        process(1)
        return c

    lax.fori_loop(0, NCH2 // 2 - 1, body, 0)
    issue(NCH2 - 1, 1)
    process(0)
    process(1)
    plsc.subcore_barrier()

    pltpu.sync_copy(acc_sp.at[pl.ds(sub * 632, 632)],
                    acc_o.at[core, pl.ds(sub * 632, 632)])
    pltpu.sync_copy(den_sp.at[pl.ds(sub * 632, 632)],
                    den_o.at[core, pl.ds(sub * 632, 632)])


_sc_l1 = pl.kernel(
    _l1_body,
    out_type=[
        jax.ShapeDtypeStruct((2, 40064, 32), jnp.float32),
        jax.ShapeDtypeStruct((10112, 16), jnp.float32),
    ],
    mesh=_MESH,
    compiler_params=pltpu.CompilerParams(use_tc_tiling_on_sc=False),
    scratch_types=[
        pltpu.VMEM_SHARED((ACC1_R, 32), jnp.float32),
        pltpu.VMEM_SHARED((DEN_R, 16), jnp.float32),
        pltpu.VMEM((K,), jnp.int32),
        pltpu.VMEM((K,), jnp.int32),
        pltpu.VMEM((K,), jnp.int32),
        pltpu.VMEM((K,), jnp.int32),
        pltpu.VMEM((K,), jnp.int32),
        pltpu.VMEM((K,), jnp.int32),
        pltpu.VMEM((K,), jnp.int32),
        pltpu.VMEM((K,), jnp.int32),
        pltpu.VMEM((K,), jnp.int32),
        pltpu.VMEM((K, 16), jnp.float32),
        pltpu.VMEM((K, 16), jnp.float32),
        pltpu.VMEM((K, 16), jnp.float32),
        pltpu.VMEM((K, 32), jnp.float32),
        pltpu.VMEM((K, 32), jnp.float32),
        pltpu.VMEM((K, 32), jnp.float32),
        pltpu.VMEM((K, 32), jnp.float32),
        pltpu.VMEM((K,), jnp.int32),
        pltpu.VMEM((K,), jnp.int32),
        pltpu.VMEM((K,), jnp.int32),
        pltpu.VMEM((K,), jnp.int32),
        pltpu.VMEM((K,), jnp.int32),
        pltpu.VMEM((K,), jnp.int32),
        pltpu.VMEM((K,), jnp.int32),
        pltpu.VMEM((K, 16), jnp.float32),
        pltpu.VMEM((K, 16), jnp.float32),
        pltpu.VMEM((K, 16), jnp.float32),
        pltpu.VMEM((K, 32), jnp.float32),
        pltpu.VMEM((K, 32), jnp.float32),
        pltpu.VMEM((K, 32), jnp.float32),
        pltpu.VMEM((K, 32), jnp.float32),
        pltpu.SemaphoreType.DMA,
        pltpu.SemaphoreType.DMA,
        pltpu.SemaphoreType.DMA,
        pltpu.SemaphoreType.DMA,
        pltpu.SemaphoreType.DMA,
        pltpu.SemaphoreType.DMA,
        pltpu.SemaphoreType.DMA,
        pltpu.SemaphoreType.DMA,
        pltpu.SemaphoreType.DMA,
        pltpu.SemaphoreType.DMA,
        pltpu.SemaphoreType.DMA,
        pltpu.SemaphoreType.DMA,
    ],
)

_sc_l2 = pl.kernel(
    _l2_body,
    out_type=[
        jax.ShapeDtypeStruct((2, 10112, 64), jnp.float32),
        jax.ShapeDtypeStruct((2, 10112, 16), jnp.float32),
    ],
    mesh=_MESH,
    compiler_params=pltpu.CompilerParams(use_tc_tiling_on_sc=False),
    scratch_types=[
        pltpu.VMEM_SHARED((ACC2_R, 64), jnp.float32),
        pltpu.VMEM_SHARED((DEN_R, 16), jnp.float32),
        pltpu.VMEM((K,), jnp.int32),
        pltpu.VMEM((K,), jnp.int32),
        pltpu.VMEM((K,), jnp.int32),
        pltpu.VMEM((K, 16), jnp.float32),
        pltpu.VMEM((K, 16), jnp.float32),
        pltpu.VMEM((K, 16), jnp.float32),
        pltpu.VMEM((K, 64), jnp.float32),
        pltpu.VMEM((K,), jnp.int32),
        pltpu.VMEM((K,), jnp.int32),
        pltpu.VMEM((K,), jnp.int32),
        pltpu.VMEM((K, 16), jnp.float32),
        pltpu.VMEM((K, 16), jnp.float32),
        pltpu.VMEM((K, 16), jnp.float32),
        pltpu.VMEM((K, 64), jnp.float32),
        pltpu.SemaphoreType.DMA,
        pltpu.SemaphoreType.DMA,
        pltpu.SemaphoreType.DMA,
        pltpu.SemaphoreType.DMA,
        pltpu.SemaphoreType.DMA,
        pltpu.SemaphoreType.DMA,
    ],
)


_BLK = 1000


def _tc1_body(x_ref, w1_ref, asp_ref, adp_ref, h_ref, a_ref, b_ref):
    h = x_ref[...] @ w1_ref[...]
    h_ref[...] = h
    a_ref[...] = h @ asp_ref[...]
    b_ref[...] = h @ adp_ref[...]


_tc1 = pl.pallas_call(
    _tc1_body,
    grid=(N // _BLK,),
    in_specs=[
        pl.BlockSpec((_BLK, IN_CH), lambda i: (i, 0)),
        pl.BlockSpec((IN_CH, HEADS * N_UNITS), lambda i: (0, 0)),
        pl.BlockSpec((HEADS * N_UNITS, 16), lambda i: (0, 0)),
        pl.BlockSpec((HEADS * N_UNITS, 16), lambda i: (0, 0)),
    ],
    out_specs=[
        pl.BlockSpec((_BLK, HEADS * N_UNITS), lambda i: (i, 0)),
        pl.BlockSpec((_BLK, 16), lambda i: (i, 0)),
        pl.BlockSpec((_BLK, 16), lambda i: (i, 0)),
    ],
    out_shape=[
        jax.ShapeDtypeStruct((N, HEADS * N_UNITS), jnp.float32),
        jax.ShapeDtypeStruct((N, 16), jnp.float32),
        jax.ShapeDtypeStruct((N, 16), jnp.float32),
    ],
)


def _tc2_body(acc_ref, den_ref, e_ref, b1_ref, w2_ref, asp_ref, adp_ref,
              h2_ref, a_ref, b_ref):
    r = 1.0 / (den_ref[...] + 1e-16)
    rexp = r @ e_ref[...]
    h2p = jnp.maximum(acc_ref[...] * rexp + b1_ref[...], 0.0)
    h2 = h2p @ w2_ref[...]
    h2_ref[...] = h2
    a_ref[...] = h2 @ asp_ref[...]
    b_ref[...] = h2 @ adp_ref[...]


_tc2 = pl.pallas_call(
    _tc2_body,
    grid=(N // _BLK,),
    in_specs=[
        pl.BlockSpec((_BLK, HEADS * N_UNITS), lambda i: (i, 0)),
        pl.BlockSpec((_BLK, 16), lambda i: (i, 0)),
        pl.BlockSpec((16, HEADS * N_UNITS), lambda i: (0, 0)),
        pl.BlockSpec((1, HEADS * N_UNITS), lambda i: (0, 0)),
        pl.BlockSpec((HEADS * N_UNITS, OUT_CH), lambda i: (0, 0)),
        pl.BlockSpec((OUT_CH, 16), lambda i: (0, 0)),
        pl.BlockSpec((OUT_CH, 16), lambda i: (0, 0)),
    ],
    out_specs=[
        pl.BlockSpec((_BLK, OUT_CH), lambda i: (i, 0)),
        pl.BlockSpec((_BLK, 16), lambda i: (i, 0)),
        pl.BlockSpec((_BLK, 16), lambda i: (i, 0)),
    ],
    out_shape=[
        jax.ShapeDtypeStruct((N, OUT_CH), jnp.float32),
        jax.ShapeDtypeStruct((N, 16), jnp.float32),
        jax.ShapeDtypeStruct((N, 16), jnp.float32),
    ],
)


def _tc3_body(a0_ref, a1_ref, d0_ref, d1_ref, b2_ref, o_ref):
    den = d0_ref[...][:, 0:1] + d1_ref[...][:, 0:1] + 1e-16
    s = (a0_ref[...] + a1_ref[...]) / den + b2_ref[...]
    m = jnp.max(s, axis=-1, keepdims=True)
    ex = jnp.exp(s - m)
    o_ref[...] = (s - m) - jnp.log(jnp.sum(ex, axis=-1, keepdims=True))


_tc3 = pl.pallas_call(
    _tc3_body,
    grid=(N // _BLK,),
    in_specs=[
        pl.BlockSpec((_BLK, OUT_CH), lambda i: (i, 0)),
        pl.BlockSpec((_BLK, OUT_CH), lambda i: (i, 0)),
        pl.BlockSpec((_BLK, 16), lambda i: (i, 0)),
        pl.BlockSpec((_BLK, 16), lambda i: (i, 0)),
        pl.BlockSpec((1, OUT_CH), lambda i: (0, 0)),
    ],
    out_specs=pl.BlockSpec((_BLK, OUT_CH), lambda i: (i, 0)),
    out_shape=jax.ShapeDtypeStruct((N, OUT_CH), jnp.float32),
)


def kernel(x, edge_index, W1, att_src1, att_dst1, b1, W2, att_src2, att_dst2, b2):
    f32 = jnp.float32
    loop = jnp.arange(N, dtype=jnp.int32)
    padi = jnp.full((EP - E - N,), N, jnp.int32)
    src = jnp.concatenate([edge_index[0], loop, padi])
    dst = jnp.concatenate([edge_index[1], loop, padi])
    epk = src * 16384 + dst

    hd = jnp.arange(HEADS)
    AsP1 = jnp.zeros((HEADS, N_UNITS, 16), f32).at[hd, :, hd].set(att_src1)
    AsP1 = AsP1.reshape(HEADS * N_UNITS, 16)
    AdP1 = jnp.zeros((HEADS, N_UNITS, 16), f32).at[hd, :, hd].set(att_dst1)
    AdP1 = AdP1.reshape(HEADS * N_UNITS, 16)
    As2P = jnp.zeros((OUT_CH, 16), f32).at[:, 0].set(att_src2[0])
    Ad2P = jnp.zeros((OUT_CH, 16), f32).at[:, 0].set(att_dst2[0])
    E16 = jnp.concatenate(
        [jnp.kron(jnp.eye(HEADS, dtype=f32), jnp.ones((1, N_UNITS), f32)),
         jnp.zeros((8, HEADS * N_UNITS), f32)], axis=0)

    h1, aS1, aD1 = _tc1(x, W1, AsP1, AdP1)

    neg = jnp.full((1, 16), -1e30, f32)

    def _dbl(a, padrow):
        rot = jnp.concatenate([a[:, 4:8], a[:, 0:4], a[:, 8:16]], axis=1)
        return jnp.concatenate([a, padrow, rot, padrow], axis=0)

    aS1t = _dbl(aS1, neg)
    aD1t = _dbl(aD1, jnp.zeros((1, 16), f32))
    h1t = jnp.concatenate(
        [h1.reshape(N * HEADS, N_UNITS),
         jnp.zeros((HEADS, N_UNITS), f32)], axis=0)

    zacc1 = jnp.zeros((632, 32), f32)
    zden = jnp.zeros((632, 16), f32)
    acc1, den1 = _sc_l1(epk, aS1t, aD1t, h1t, zacc1, zden)
    acc1 = acc1[:, :N * 4]
    den1 = den1[:N]

    acc1c = jnp.concatenate([acc1[0].reshape(N, 4 * N_UNITS),
                             acc1[1].reshape(N, 4 * N_UNITS)], axis=1)

    h2, aS2, aD2 = _tc2(acc1c, den1, E16, b1.reshape(1, -1), W2, As2P, Ad2P)

    aS2t = jnp.concatenate([aS2, neg], axis=0)
    aD2t = jnp.concatenate([aD2, jnp.zeros((1, 16), f32)], axis=0)
    h2t = jnp.concatenate([h2, jnp.zeros((1, OUT_CH), f32)], axis=0)

    zacc2 = jnp.zeros((632, 64), f32)
    acc2, den2 = _sc_l2(epk, aS2t, aD2t, h2t, zacc2, zden)
    acc2 = acc2[:, :N]
    den2 = den2[:, :N]

    return _tc3(acc2[0], acc2[1], den2[0], den2[1], b2.reshape(1, -1))
